# baseline (speedup 1.0000x reference)
import functools

import jax
import jax.numpy as jnp
import numpy as np
from jax import lax
from jax.experimental import pallas as pl
from jax.experimental.pallas import tpu as pltpu
from jax.experimental.pallas import tpu_sc as plsc

_PS = 4
_GRID = 32
_AIR = (102, 576, 3352)
_HREG = 3728
_NHIST = 4

_NC, _NS = 2, 16
_NW = _NC * _NS
_L = 64

_LOGTAB = np.zeros(80, np.float32)
_LOGTAB[1:65] = np.log(np.arange(1, 65, dtype=np.float64)).astype(np.float32)


def _sc_body(flat_hbm, logtab_hbm, out_hbm, data_v, hist_v, hist2_v,
             logtab_v, out_v, dma_sem, dma_sem2):
    pw = data_v.shape[0]
    wid = lax.axis_index("c") * _NS + lax.axis_index("s")

    hb = pw // 2
    slab0 = pltpu.async_copy(flat_hbm.at[pl.ds(wid * pw, hb)],
                             data_v.at[pl.ds(0, hb)], dma_sem)
    slab1 = pltpu.async_copy(flat_hbm.at[pl.ds(wid * pw + hb, hb)],
                             data_v.at[pl.ds(hb, hb)], dma_sem2)
    pltpu.sync_copy(logtab_hbm, logtab_v)

    zeros16 = jnp.zeros((16,), jnp.int32)
    zeros16f = jnp.zeros((16,), jnp.float32)
    ones16 = jnp.ones((16,), jnp.int32)
    full64 = jnp.full((16,), _L, jnp.int32)

    def zero_body(j, carry):
        for u in range(8):
            hist_v[pl.ds(j * 256 + u * 32, 16)] = zeros16
            hist_v[pl.ds(j * 256 + u * 32 + 16, 16)] = zeros16
            hist2_v[pl.ds(j * 256 + u * 32, 16)] = zeros16
            hist2_v[pl.ds(j * 256 + u * 32 + 16, 16)] = zeros16
        return carry
    nz = _NHIST * _HREG
    lax.fori_loop(0, nz // 256, zero_body, 0)
    for u in range(nz % 256 // 16):
        hist_v[pl.ds(nz // 256 * 256 + u * 16, 16)] = zeros16
        hist2_v[pl.ds(nz // 256 * 256 + u * 16, 16)] = zeros16
    slab0.wait()

    lane = lax.iota(jnp.int32, 16)
    pat = (lane >> 2) * _HREG
    airp = [pat + a for a in _AIR]

    def load_unit(rows, h):
        return [data_v[pl.ds(r + 16 * h, 16)] + pat for r in rows]

    def scatter_unit(hist, idxs):
        for idx in idxs:
            plsc.addupdate_scatter(hist, [idx], ones16)

    def gather_unit(hist, idxs, acc):
        n_t = [plsc.load_gather(hist, [a]) for a in airp]
        s_vec = full64 - (n_t[0] + n_t[1] + n_t[2])
        log_s = plsc.load_gather(logtab_v, [s_vec])
        recip = 1.0 / jnp.maximum(s_vec.astype(jnp.float32), 1.0)
        corr = zeros16f
        for n in n_t:
            log_n = plsc.load_gather(logtab_v, [n])
            corr = corr + n.astype(jnp.float32) * (log_s - log_n)
        lcs = []
        for idx in idxs:
            cv = plsc.load_gather(hist, [idx])
            lcs.append(plsc.load_gather(logtab_v, [cv]))
        while len(lcs) > 1:
            lcs = [a + b for a, b in zip(lcs[::2], lcs[1::2])]
        inner = log_s * 16.0 - lcs[0] - corr * 0.25
        return acc + inner * recip

    def clear_unit(hist, idxs):
        for idx in idxs:
            plsc.store_scatter(hist, [idx], zeros16)

    def group_body(g, acc):
        base = ((g >> 6) * 32768 + ((g >> 3) & 7) * 4096 + (g & 7) * 128)
        rows = [base + a * 1024 + c * 32 for a in range(_PS)
                for c in range(_PS)]
        lo = load_unit(rows, 0)
        scatter_unit(hist_v, lo)
        hi = load_unit(rows, 1)
        scatter_unit(hist2_v, hi)
        acc = gather_unit(hist_v, lo, acc)
        clear_unit(hist_v, lo)
        acc = gather_unit(hist2_v, hi, acc)
        clear_unit(hist2_v, hi)
        return acc

    n_groups = pw // (16 * 32)

    def pair_body(p, acc):
        return group_body(2 * p + 1, group_body(2 * p, acc))

    acc = lax.fori_loop(0, n_groups // 4, pair_body, zeros16f)
    slab1.wait()
    acc = lax.fori_loop(n_groups // 4, n_groups // 2, pair_body, acc)
    out_v[...] = acc
    pltpu.sync_copy(out_v, out_hbm.at[wid])


@jax.jit
def _sc_entropy(flat, logtab):
    pw = flat.shape[0] // _NW
    fn = functools.partial(
        pl.kernel,
        out_type=jax.ShapeDtypeStruct((_NW, 16), jnp.float32),
        mesh=plsc.VectorSubcoreMesh(
            core_axis_name="c", subcore_axis_name="s",
            num_cores=_NC, num_subcores=_NS),
        scratch_types=[
            pltpu.VMEM((pw,), jnp.int32),
            pltpu.VMEM((_NHIST * _HREG,), jnp.int32),
            pltpu.VMEM((_NHIST * _HREG,), jnp.int32),
            pltpu.VMEM((80,), jnp.float32),
            pltpu.VMEM((16,), jnp.float32),
            pltpu.SemaphoreType.DMA,
            pltpu.SemaphoreType.DMA,
        ],
        compiler_params=pltpu.CompilerParams(needs_layout_passes=False),
    )(_sc_body)
    return fn(flat, logtab)


def kernel(structure):
    B = structure.shape[0]
    n = _GRID // _PS
    num_patches = n * n * n
    partials = _sc_entropy(structure.reshape(-1), jnp.asarray(_LOGTAB))
    total = jnp.sum(partials)
    return total / (B * num_patches + 1e-06)


# speedup vs baseline: 1.1828x; 1.1828x over previous
import functools

import jax
import jax.numpy as jnp
import numpy as np
from jax import lax
from jax.experimental import pallas as pl
from jax.experimental.pallas import tpu as pltpu
from jax.experimental.pallas import tpu_sc as plsc

_PS = 4
_GRID = 32
_AIR = (102, 576, 3352)
_HREG = 3728
_NHIST = 4

_NC, _NS = 2, 16
_NW = _NC * _NS
_L = 64

_LOGTAB = np.zeros(80, np.float32)
_LOGTAB[1:65] = np.log(np.arange(1, 65, dtype=np.float64)).astype(np.float32)


def _sc_body(flat_hbm, logtab_hbm, out_hbm, data_v, hist_v, hist2_v,
             logtab_v, out_v, dma_sem):
    pw = data_v.shape[0]
    wid = lax.axis_index("c") * _NS + lax.axis_index("s")

    slab = pltpu.async_copy(flat_hbm.at[pl.ds(wid * pw, pw)], data_v,
                            dma_sem)
    pltpu.sync_copy(logtab_hbm, logtab_v)

    zeros16 = jnp.zeros((16,), jnp.int32)
    zeros16f = jnp.zeros((16,), jnp.float32)
    ones16 = jnp.ones((16,), jnp.int32)
    full64 = jnp.full((16,), _L, jnp.int32)

    def zero_body(j, carry):
        for u in range(8):
            hist_v[pl.ds(j * 256 + u * 32, 16)] = zeros16
            hist_v[pl.ds(j * 256 + u * 32 + 16, 16)] = zeros16
            hist2_v[pl.ds(j * 256 + u * 32, 16)] = zeros16
            hist2_v[pl.ds(j * 256 + u * 32 + 16, 16)] = zeros16
        return carry
    nz = _NHIST * _HREG
    lax.fori_loop(0, nz // 256, zero_body, 0)
    for u in range(nz % 256 // 16):
        hist_v[pl.ds(nz // 256 * 256 + u * 16, 16)] = zeros16
        hist2_v[pl.ds(nz // 256 * 256 + u * 16, 16)] = zeros16
    slab.wait()

    lane = lax.iota(jnp.int32, 16)
    pat = (lane >> 2) * _HREG
    airp = [pat + a for a in _AIR]

    def load_unit(rows, h):
        return [data_v[pl.ds(r + 16 * h, 16)] + pat for r in rows]

    def scatter_unit(hist, idxs):
        for idx in idxs:
            plsc.addupdate_scatter(hist, [idx], ones16)

    def gather_unit(hist, idxs, acc):
        n_t = [plsc.load_gather(hist, [a]) for a in airp]
        s_vec = full64 - (n_t[0] + n_t[1] + n_t[2])
        log_s = plsc.load_gather(logtab_v, [s_vec])
        recip = 1.0 / jnp.maximum(s_vec.astype(jnp.float32), 1.0)
        corr = zeros16f
        for n in n_t:
            log_n = plsc.load_gather(logtab_v, [n])
            corr = corr + n.astype(jnp.float32) * (log_s - log_n)
        lcs = []
        for idx in idxs:
            cv = plsc.load_gather(hist, [idx])
            lcs.append(plsc.load_gather(logtab_v, [cv]))
        while len(lcs) > 1:
            lcs = [a + b for a, b in zip(lcs[::2], lcs[1::2])]
        inner = log_s * 16.0 - lcs[0] - corr * 0.25
        return acc + inner * recip

    def clear_unit(hist, idxs):
        for idx in idxs:
            plsc.store_scatter(hist, [idx], zeros16)

    def group_body(g, acc):
        base = ((g >> 6) * 32768 + ((g >> 3) & 7) * 4096 + (g & 7) * 128)
        rows = [base + a * 1024 + c * 32 for a in range(_PS)
                for c in range(_PS)]
        lo = load_unit(rows, 0)
        scatter_unit(hist_v, lo)
        hi = load_unit(rows, 1)
        scatter_unit(hist2_v, hi)
        acc = gather_unit(hist_v, lo, acc)
        clear_unit(hist_v, lo)
        acc = gather_unit(hist2_v, hi, acc)
        clear_unit(hist2_v, hi)
        return acc

    n_groups = pw // (16 * 32)
    acc = lax.fori_loop(0, n_groups, group_body, zeros16f)
    out_v[...] = acc
    pltpu.sync_copy(out_v, out_hbm.at[wid])


@jax.jit
def _sc_entropy(flat, logtab):
    pw = flat.shape[0] // _NW
    fn = functools.partial(
        pl.kernel,
        out_type=jax.ShapeDtypeStruct((_NW, 16), jnp.float32),
        mesh=plsc.VectorSubcoreMesh(
            core_axis_name="c", subcore_axis_name="s",
            num_cores=_NC, num_subcores=_NS),
        scratch_types=[
            pltpu.VMEM((pw,), jnp.int32),
            pltpu.VMEM((_NHIST * _HREG,), jnp.int32),
            pltpu.VMEM((_NHIST * _HREG,), jnp.int32),
            pltpu.VMEM((80,), jnp.float32),
            pltpu.VMEM((16,), jnp.float32),
            pltpu.SemaphoreType.DMA,
        ],
        compiler_params=pltpu.CompilerParams(needs_layout_passes=False),
    )(_sc_body)
    return fn(flat, logtab)


def kernel(structure):
    B = structure.shape[0]
    n = _GRID // _PS
    num_patches = n * n * n
    partials = _sc_entropy(structure.reshape(-1), jnp.asarray(_LOGTAB))
    total = jnp.sum(partials)
    return total / (B * num_patches + 1e-06)
